# Initial kernel scaffold; baseline (speedup 1.0000x reference)
import jax
import jax.numpy as jnp
from jax.experimental import pallas as pl


def kernel(x, lens_effects, diskernel, lens_mask):
    raise NotImplementedError("write your pallas kernel here")


# speedup vs baseline: 10.9764x; 10.9764x over previous
import numpy as np
import jax
import jax.numpy as jnp
from jax.experimental import pallas as pl
from jax.experimental.pallas import tpu as pltpu

TILE_H = 8


def _tap_table(lens):
    r = lens // 2
    ys, xs = np.meshgrid(np.arange(lens) - r, np.arange(lens) - r,
                         indexing='ij')
    d = np.sqrt(ys.astype(np.float64) ** 2 + xs.astype(np.float64) ** 2)
    d32 = d.astype(np.float32)
    mask = d32 <= r + 1e-6
    return [(dy, dx, float(d32[dy, dx]))
            for dy in range(lens) for dx in range(lens) if mask[dy, dx]]


def _prep_body(le_ref, disp_ref, u_ref):
    b = pl.program_id(0)
    u_ref[...] = jnp.abs(disp_ref[...]) * le_ref[b, 0]


WIN = TILE_H + 16


def _main_body(taps, u_ref, p_ref, out_ref):
    t = pl.program_id(1)
    r0 = pl.multiple_of(t * TILE_H, TILE_H)
    u_win = u_ref[0, pl.ds(r0, WIN), :]
    p_win = p_ref[0, :, pl.ds(r0, WIN), :]
    acc = jnp.zeros((3, TILE_H, 512), jnp.float32)
    wsum = jnp.zeros((TILE_H, 512), jnp.float32)
    for dy, dx, dval in taps:
        usl = u_win[dy:dy + TILE_H, dx:dx + 512]
        cov = jnp.minimum(jnp.maximum((usl - dval) + 0.5, 0.0), 1.0)
        a = p_win[3, dy:dy + TILE_H, dx:dx + 512]
        w = cov * a
        rgb = p_win[:3, dy:dy + TILE_H, dx:dx + 512]
        acc = acc + w[None, :, :] * rgb
        wsum = wsum + w
    out_ref[...] = (acc / (wsum + 1e-6)[None, :, :])[None]


def kernel(x, lens_effects, diskernel, lens_mask):
    b, c, h, w = x.shape
    lens = diskernel.shape[0]
    pad = lens // 2
    taps = _tap_table(lens)

    disp = x[:, 4]
    u = pl.pallas_call(
        _prep_body,
        grid=(b,),
        in_specs=[
            pl.BlockSpec(memory_space=pltpu.SMEM),
            pl.BlockSpec((1, h, w), lambda i: (i, 0, 0)),
        ],
        out_specs=pl.BlockSpec((1, h, w), lambda i: (i, 0, 0)),
        out_shape=jax.ShapeDtypeStruct((b, h, w), jnp.float32),
    )(lens_effects, disp)

    hp = h + 2 * pad
    wp = w + 2 * pad
    hp8 = ((hp + 7) // 8) * 8
    wp128 = ((wp + 127) // 128) * 128

    u_pad = jnp.pad(u, ((0, 0), (pad, pad), (pad, pad)), mode='edge')
    u_pad = jnp.pad(u_pad, ((0, 0), (0, hp8 - hp), (0, wp128 - wp)))
    rgba = x[:, :4]
    p_pad = jnp.pad(rgba, ((0, 0), (0, 0), (pad, pad), (pad, pad)),
                    mode='edge')
    p_pad = jnp.pad(p_pad, ((0, 0), (0, 0), (0, hp8 - hp), (0, wp128 - wp)))

    out = pl.pallas_call(
        lambda u_ref, p_ref, o_ref: _main_body(taps, u_ref, p_ref, o_ref),
        grid=(b, h // TILE_H),
        in_specs=[
            pl.BlockSpec((1, hp8, wp128), lambda i, t: (i, 0, 0)),
            pl.BlockSpec((1, 4, hp8, wp128), lambda i, t: (i, 0, 0, 0)),
        ],
        out_specs=pl.BlockSpec((1, 3, TILE_H, w), lambda i, t: (i, 0, t, 0)),
        out_shape=jax.ShapeDtypeStruct((b, 3, h, w), jnp.float32),
    )(u_pad, p_pad)
    return out
